# Initial kernel scaffold; baseline (speedup 1.0000x reference)
import jax
import jax.numpy as jnp
from jax.experimental import pallas as pl


def kernel(loc, conf, landms, prior_box):
    raise NotImplementedError("write your pallas kernel here")


# speedup vs baseline: 24.5071x; 24.5071x over previous
import functools

import jax
import jax.numpy as jnp
from jax import lax
from jax.experimental import pallas as pl
from jax.experimental.pallas import tpu as pltpu
from jax.experimental.pallas import tpu_sc as plsc

_VAR0, _VAR1 = 0.1, 0.2
_CONF_THR = 0.5
_NMS_THR = 0.4
_TOP_K = 750
_NP = 20000
_PP = 20480
_R, _L = 160, 128
_SIZE = 640.0
_BIG = 2 ** 30
_KB = 768
_KD = 128


def _decode_nms_body(in_ref, comp_ref, ki_ref):
    lx, ly, lw, lh = in_ref[0], in_ref[1], in_ref[2], in_ref[3]
    pcx, pcy, pw, ph = in_ref[4], in_ref[5], in_ref[6], in_ref[7]
    conf1 = in_ref[8]

    cx = pcx + lx * _VAR0 * pw
    cy = pcy + ly * _VAR0 * ph
    w = pw * jnp.exp(lw * _VAR1)
    h = ph * jnp.exp(lh * _VAR1)
    x1 = (cx - w / 2.0) * _SIZE
    y1 = (cy - h / 2.0) * _SIZE
    x2 = (cx + w / 2.0) * _SIZE
    y2 = (cy + h / 2.0) * _SIZE
    comp_ref[0] = x1
    comp_ref[1] = y1
    comp_ref[2] = x2
    comp_ref[3] = y2
    for k in range(5):
        comp_ref[4 + 2 * k] = (pcx + pw * in_ref[9 + 2 * k] * _VAR0) * _SIZE
        comp_ref[5 + 2 * k] = (pcy + ph * in_ref[9 + 2 * k + 1] * _VAR0) * _SIZE

    area = jnp.maximum(x2 - x1, 0.0) * jnp.maximum(y2 - y1, 0.0)
    alive0 = jnp.where(conf1 > _CONF_THR, 1.0, 0.0)

    iota2 = (lax.broadcasted_iota(jnp.int32, (_R, _L), 0) * _L
             + lax.broadcasted_iota(jnp.int32, (_R, _L), 1))
    lane_iota = lax.broadcasted_iota(jnp.int32, (1, _L), 1)
    kio = (lax.broadcasted_iota(jnp.int32, (8, 128), 0) * 128
           + lax.broadcasted_iota(jnp.int32, (8, 128), 1))

    def body(s, carry):
        alive, keep = carry
        masked = jnp.where(alive > 0.0, iota2, _BIG)
        first = jnp.min(masked)
        valid = first < _PP
        fs = jnp.where(valid, first, 0)
        row = fs // _L
        lane = fs - row * _L

        def ext(c):
            v = comp_ref[c, pl.ds(row, 1), :]
            return jnp.sum(jnp.where(lane_iota == lane, v, 0.0))

        xi1, yi1, xi2, yi2 = ext(0), ext(1), ext(2), ext(3)
        ai = jnp.maximum(xi2 - xi1, 0.0) * jnp.maximum(yi2 - yi1, 0.0)
        bx1 = jnp.maximum(xi1, x1)
        by1 = jnp.maximum(yi1, y1)
        bx2 = jnp.minimum(xi2, x2)
        by2 = jnp.minimum(yi2, y2)
        inter = jnp.maximum(bx2 - bx1, 0.0) * jnp.maximum(by2 - by1, 0.0)
        iou = inter / (ai + area - inter + 1e-9)
        kill = (iou > _NMS_THR) | (iota2 == first)
        alive = jnp.where(jnp.logical_and(valid, kill), 0.0, alive)
        keep = jnp.where(kio == s, jnp.where(valid, first, _NP), keep)
        return alive, keep

    _, keep = lax.fori_loop(0, _TOP_K, body, (alive0, jnp.full((8, 128), _NP, jnp.int32)))
    ki_ref[...] = keep


def _sc_gather(table, idx):
    info = plsc.get_sparse_core_info()
    nw = info.num_cores * info.num_subcores
    bpw = _KB // nw
    mesh = plsc.VectorSubcoreMesh(core_axis_name="c", subcore_axis_name="s")

    @functools.partial(
        pl.kernel, mesh=mesh,
        out_type=jax.ShapeDtypeStruct((_KB, _KD), jnp.float32),
        scratch_types=[
            pltpu.VMEM((bpw,), jnp.int32),
            pltpu.VMEM((bpw, _KD), jnp.float32),
            pltpu.SemaphoreType.DMA,
        ],
    )
    def k(table_hbm, idx_hbm, out_hbm, idx_v, rows_v, sem):
        wid = lax.axis_index("s") * info.num_cores + lax.axis_index("c")
        base = wid * bpw
        pltpu.sync_copy(idx_hbm.at[pl.ds(base, bpw)], idx_v)
        pltpu.async_copy(table_hbm.at[idx_v], rows_v, sem).wait()
        pltpu.sync_copy(rows_v, out_hbm.at[pl.ds(base, bpw)])

    return k(table, idx)


def _comps(a):
    return a.T.reshape(a.shape[1], _R, _L)


def kernel(loc, conf, landms, prior_box):
    pad = _PP - _NP
    locp = jnp.pad(loc[0], ((0, pad), (0, 0)))
    confp = jnp.pad(conf[0], ((0, pad), (0, 0)))
    lmp = jnp.pad(landms[0], ((0, pad), (0, 0)))
    prp = jnp.pad(prior_box, ((0, pad), (0, 0)))
    confT = _comps(confp)
    inp = jnp.concatenate(
        [_comps(locp), _comps(prp), confT[1:2], _comps(lmp)], axis=0)

    comp, ki = pl.pallas_call(
        _decode_nms_body,
        out_shape=[
            jax.ShapeDtypeStruct((14, _R, _L), jnp.float32),
            jax.ShapeDtypeStruct((8, 128), jnp.int32),
        ],
    )(inp)

    table = jnp.concatenate(
        [confT, comp, jnp.zeros((_KD - 16, _R, _L), jnp.float32)],
        axis=0).reshape(_KD, _PP).T
    idx = ki.reshape(-1)[:_KB]
    rows = _sc_gather(table, idx)
    conf_out = rows[:_TOP_K, 0:2]
    loc_out = rows[:_TOP_K, 2:6]
    lm_out = rows[:_TOP_K, 6:16]
    return conf_out, lm_out, loc_out


# speedup vs baseline: 26.0547x; 1.0632x over previous
import functools

import jax
import jax.numpy as jnp
from jax import lax
from jax.experimental import pallas as pl
from jax.experimental.pallas import tpu as pltpu
from jax.experimental.pallas import tpu_sc as plsc

_VAR0, _VAR1 = 0.1, 0.2
_CONF_THR = 0.5
_NMS_THR = 0.4
_TOP_K = 750
_NP = 20000
_PP = 20480
_R, _L = 160, 128
_SIZE = 640.0
_BIG = 2 ** 30
_KB = 768
_KD = 128


def _decode_nms_body(in_ref, comp_ref, ki_ref):
    lx, ly, lw, lh = in_ref[0], in_ref[1], in_ref[2], in_ref[3]
    pcx, pcy, pw, ph = in_ref[4], in_ref[5], in_ref[6], in_ref[7]
    conf1 = in_ref[8]

    cx = pcx + lx * _VAR0 * pw
    cy = pcy + ly * _VAR0 * ph
    w = pw * jnp.exp(lw * _VAR1)
    h = ph * jnp.exp(lh * _VAR1)
    x1 = (cx - w / 2.0) * _SIZE
    y1 = (cy - h / 2.0) * _SIZE
    x2 = (cx + w / 2.0) * _SIZE
    y2 = (cy + h / 2.0) * _SIZE
    comp_ref[0] = x1
    comp_ref[1] = y1
    comp_ref[2] = x2
    comp_ref[3] = y2
    for k in range(5):
        comp_ref[4 + 2 * k] = (pcx + pw * in_ref[9 + 2 * k] * _VAR0) * _SIZE
        comp_ref[5 + 2 * k] = (pcy + ph * in_ref[9 + 2 * k + 1] * _VAR0) * _SIZE

    area = jnp.maximum(x2 - x1, 0.0) * jnp.maximum(y2 - y1, 0.0)
    c_thr = _NMS_THR / (1.0 + _NMS_THR)
    ta = area * c_thr

    iota2 = (lax.broadcasted_iota(jnp.int32, (_R, _L), 0) * _L
             + lax.broadcasted_iota(jnp.int32, (_R, _L), 1))
    lane_iota = lax.broadcasted_iota(jnp.int32, (1, _L), 1)
    kio = (lax.broadcasted_iota(jnp.int32, (8, 128), 0) * 128
           + lax.broadcasted_iota(jnp.int32, (8, 128), 1))

    miota0 = jnp.where(conf1 > _CONF_THR, iota2, _BIG)
    first0 = jnp.min(miota0)

    def cond(carry):
        s, first, _, _ = carry
        return jnp.logical_and(s < _TOP_K, first < _PP)

    def body(carry):
        s, first, miota, keep = carry
        row = first // _L
        lane = first - row * _L

        def ext(c):
            v = comp_ref[c, pl.ds(row, 1), :]
            return jnp.sum(jnp.where(lane_iota == lane, v, 0.0))

        xi1, yi1, xi2, yi2 = ext(0), ext(1), ext(2), ext(3)
        ai = jnp.maximum(xi2 - xi1, 0.0) * jnp.maximum(yi2 - yi1, 0.0)
        bx1 = jnp.maximum(xi1, x1)
        by1 = jnp.maximum(yi1, y1)
        bx2 = jnp.minimum(xi2, x2)
        by2 = jnp.minimum(yi2, y2)
        inter = jnp.maximum(bx2 - bx1, 0.0) * jnp.maximum(by2 - by1, 0.0)
        tai = ai * c_thr + (1e-9 * _NMS_THR)
        kill = (inter > ta + tai) | (miota == first)
        miota = jnp.where(kill, _BIG, miota)
        keep = jnp.where(kio == s, first, keep)
        return s + 1, jnp.min(miota), miota, keep

    _, _, _, keep = lax.while_loop(
        cond, body,
        (0, first0, miota0, jnp.full((8, 128), _NP, jnp.int32)))
    ki_ref[...] = keep


def _sc_gather(table, idx):
    info = plsc.get_sparse_core_info()
    nw = info.num_cores * info.num_subcores
    bpw = _KB // nw
    mesh = plsc.VectorSubcoreMesh(core_axis_name="c", subcore_axis_name="s")

    @functools.partial(
        pl.kernel, mesh=mesh,
        out_type=jax.ShapeDtypeStruct((_KB, _KD), jnp.float32),
        scratch_types=[
            pltpu.VMEM((bpw,), jnp.int32),
            pltpu.VMEM((bpw, _KD), jnp.float32),
            pltpu.SemaphoreType.DMA,
        ],
    )
    def k(table_hbm, idx_hbm, out_hbm, idx_v, rows_v, sem):
        wid = lax.axis_index("s") * info.num_cores + lax.axis_index("c")
        base = wid * bpw
        pltpu.sync_copy(idx_hbm.at[pl.ds(base, bpw)], idx_v)
        pltpu.async_copy(table_hbm.at[idx_v], rows_v, sem).wait()
        pltpu.sync_copy(rows_v, out_hbm.at[pl.ds(base, bpw)])

    return k(table, idx)


def _comps(a):
    return a.T.reshape(a.shape[1], _R, _L)


def kernel(loc, conf, landms, prior_box):
    pad = _PP - _NP
    locp = jnp.pad(loc[0], ((0, pad), (0, 0)))
    confp = jnp.pad(conf[0], ((0, pad), (0, 0)))
    lmp = jnp.pad(landms[0], ((0, pad), (0, 0)))
    prp = jnp.pad(prior_box, ((0, pad), (0, 0)))
    confT = _comps(confp)
    inp = jnp.concatenate(
        [_comps(locp), _comps(prp), confT[1:2], _comps(lmp)], axis=0)

    comp, ki = pl.pallas_call(
        _decode_nms_body,
        out_shape=[
            jax.ShapeDtypeStruct((14, _R, _L), jnp.float32),
            jax.ShapeDtypeStruct((8, 128), jnp.int32),
        ],
    )(inp)

    table = jnp.concatenate(
        [confT, comp, jnp.zeros((_KD - 16, _R, _L), jnp.float32)],
        axis=0).reshape(_KD, _PP).T
    idx = ki.reshape(-1)[:_KB]
    rows = _sc_gather(table, idx)
    conf_out = rows[:_TOP_K, 0:2]
    loc_out = rows[:_TOP_K, 2:6]
    lm_out = rows[:_TOP_K, 6:16]
    return conf_out, lm_out, loc_out
